# Initial kernel scaffold; baseline (speedup 1.0000x reference)
import functools

import jax
import jax.numpy as jnp
from jax import lax
from jax.experimental import pallas as pl
from jax.experimental.pallas import tpu as pltpu
from jax.experimental.pallas import tpu_sc as plsc

_LANES = 16


def _embed_kernel(n_tokens, d, seq_vocab):
    info = plsc.get_sparse_core_info()
    nc, ns = info.num_cores, info.num_subcores
    nw = nc * ns
    chunk = 128
    per_w = n_tokens // nw
    n_chunks = per_w // chunk
    assert per_w % chunk == 0 and n_tokens % nw == 0

    mesh = plsc.VectorSubcoreMesh(core_axis_name="c", subcore_axis_name="s")

    @functools.partial(
        pl.kernel,
        mesh=mesh,
        out_type=jax.ShapeDtypeStruct((n_tokens, d), jnp.float32),
        scratch_types=[
            pltpu.VMEM((chunk,), jnp.int32),
            pltpu.VMEM((chunk,), jnp.int32),
            pltpu.VMEM((chunk,), jnp.int32),
            pltpu.VMEM((chunk,), jnp.int32),
            pltpu.VMEM((chunk, d), jnp.float32),
            pltpu.VMEM((chunk, d), jnp.float32),
            pltpu.SemaphoreType.DMA,
            pltpu.SemaphoreType.DMA,
        ],
    )
    def k(ids_hbm, seg_hbm, pos_hbm, tok_tab, comb_tab, out_hbm,
          idx_tok, seg_v, pos_v, cidx, rows_a, rows_b, sem_a, sem_b):
        wid = lax.axis_index("s") * nc + lax.axis_index("c")
        base = wid * per_w

        def body(g, carry):
            off = base + g * chunk
            pltpu.sync_copy(ids_hbm.at[pl.ds(off, chunk)], idx_tok)
            pltpu.sync_copy(seg_hbm.at[pl.ds(off, chunk)], seg_v)
            pltpu.sync_copy(pos_hbm.at[pl.ds(off, chunk)], pos_v)
            for i in range(chunk // _LANES):
                sl = pl.ds(i * _LANES, _LANES)
                cidx[sl] = seg_v[sl] * seq_vocab + pos_v[sl]
            ca = pltpu.async_copy(tok_tab.at[idx_tok], rows_a, sem_a)
            cb = pltpu.async_copy(comb_tab.at[cidx], rows_b, sem_b)
            ca.wait()
            cb.wait()
            for t in range(chunk):
                for j in range(d // _LANES):
                    sl = pl.ds(j * _LANES, _LANES)
                    plsc.addupdate(rows_a.at[t, sl], rows_b[t, sl])
            pltpu.sync_copy(rows_a, out_hbm.at[pl.ds(off, chunk)])
            return carry

        lax.fori_loop(0, n_chunks, body, 0)

    return k


def kernel(input_ids, segment_ids, position_ids, token_table,
           segment_table, position_table):
    b, s = input_ids.shape
    d = token_table.shape[1]
    seq_vocab = position_table.shape[0]
    n = b * s

    if position_ids is None:
        position_ids = jnp.broadcast_to(
            jnp.arange(s, dtype=input_ids.dtype)[None, :], (b, s))

    ids = input_ids.reshape(n).astype(jnp.int32)
    seg = segment_ids.reshape(n).astype(jnp.int32)
    pos = position_ids.reshape(n).astype(jnp.int32)
    comb = (segment_table[:, None, :] + position_table[None, :, :]).reshape(
        segment_table.shape[0] * seq_vocab, d)

    out = _embed_kernel(n, d, seq_vocab)(ids, seg, pos, token_table, comb)
    return out.reshape(b, s, d)


# speedup vs baseline: 4.5912x; 4.5912x over previous
import functools

import jax
import jax.numpy as jnp
from jax import lax
from jax.experimental import pallas as pl
from jax.experimental.pallas import tpu as pltpu
from jax.experimental.pallas import tpu_sc as plsc

_LANES = 16


def _embed_kernel(n_tokens, d, seq_vocab):
    info = plsc.get_sparse_core_info()
    nc, ns = info.num_cores, info.num_subcores
    nw = nc * ns
    chunk = 128
    per_w = n_tokens // nw
    n_chunks = per_w // chunk
    assert per_w % chunk == 0 and n_tokens % nw == 0

    mesh = plsc.VectorSubcoreMesh(core_axis_name="c", subcore_axis_name="s")

    @functools.partial(
        pl.kernel,
        mesh=mesh,
        out_type=jax.ShapeDtypeStruct((n_tokens, d), jnp.float32),
        scratch_types=[
            pltpu.VMEM((chunk,), jnp.int32),
            pltpu.VMEM((chunk,), jnp.int32),
            pltpu.VMEM((chunk,), jnp.int32),
            pltpu.VMEM((chunk,), jnp.int32),
            pltpu.VMEM((chunk, d), jnp.float32),
            pltpu.VMEM((chunk, d), jnp.float32),
            pltpu.SemaphoreType.DMA,
            pltpu.SemaphoreType.DMA,
        ],
        compiler_params=pltpu.CompilerParams(use_tc_tiling_on_sc=False),
    )
    def k(ids_hbm, seg_hbm, pos_hbm, tok_tab, comb_tab, out_hbm,
          idx_tok, seg_v, pos_v, cidx, rows_a, rows_b, sem_a, sem_b):
        wid = lax.axis_index("s") * nc + lax.axis_index("c")
        base = wid * per_w

        def body(g, carry):
            off = base + g * chunk
            pltpu.sync_copy(ids_hbm.at[pl.ds(off, chunk)], idx_tok)
            pltpu.sync_copy(seg_hbm.at[pl.ds(off, chunk)], seg_v)
            pltpu.sync_copy(pos_hbm.at[pl.ds(off, chunk)], pos_v)
            for i in range(chunk // _LANES):
                sl = pl.ds(i * _LANES, _LANES)
                cidx[sl] = seg_v[sl] * seq_vocab + pos_v[sl]
            ca = pltpu.async_copy(tok_tab.at[idx_tok], rows_a, sem_a)
            cb = pltpu.async_copy(comb_tab.at[cidx], rows_b, sem_b)
            ca.wait()
            cb.wait()
            for t in range(chunk):
                for j in range(d // _LANES):
                    sl = pl.ds(j * _LANES, _LANES)
                    plsc.addupdate(rows_a.at[t, sl], rows_b[t, sl])
            pltpu.sync_copy(rows_a, out_hbm.at[pl.ds(off, chunk)])
            return carry

        lax.fori_loop(0, n_chunks, body, 0)

    return k


def kernel(input_ids, segment_ids, position_ids, token_table,
           segment_table, position_table):
    b, s = input_ids.shape
    d = token_table.shape[1]
    seq_vocab = position_table.shape[0]
    n = b * s

    if position_ids is None:
        position_ids = jnp.broadcast_to(
            jnp.arange(s, dtype=input_ids.dtype)[None, :], (b, s))

    ids = input_ids.reshape(n).astype(jnp.int32)
    seg = segment_ids.reshape(n).astype(jnp.int32)
    pos = position_ids.reshape(n).astype(jnp.int32)
    comb = (segment_table[:, None, :] + position_table[None, :, :]).reshape(
        segment_table.shape[0] * seq_vocab, d)

    out = _embed_kernel(n, d, seq_vocab)(ids, seg, pos, token_table, comb)
    return out.reshape(b, s, d)


# speedup vs baseline: 5.3777x; 1.1713x over previous
import functools

import jax
import jax.numpy as jnp
from jax import lax
from jax.experimental import pallas as pl
from jax.experimental.pallas import tpu as pltpu
from jax.experimental.pallas import tpu_sc as plsc

_LANES = 16


def _embed_kernel(n_tokens, d, seq_vocab):
    info = plsc.get_sparse_core_info()
    nc, ns = info.num_cores, info.num_subcores
    nw = nc * ns
    chunk = 128
    per_w = n_tokens // nw
    n_chunks = per_w // chunk
    assert per_w % chunk == 0 and n_tokens % nw == 0

    mesh = plsc.VectorSubcoreMesh(core_axis_name="c", subcore_axis_name="s")

    @functools.partial(
        pl.kernel,
        mesh=mesh,
        out_type=jax.ShapeDtypeStruct((n_tokens, d), jnp.float32),
        scratch_types=[
            pltpu.VMEM((chunk,), jnp.int32),
            pltpu.VMEM((chunk,), jnp.int32),
            pltpu.VMEM((chunk,), jnp.int32),
            pltpu.VMEM((chunk,), jnp.int32),
            pltpu.VMEM((chunk, d), jnp.float32),
            pltpu.VMEM((chunk, d), jnp.float32),
            pltpu.SemaphoreType.DMA,
            pltpu.SemaphoreType.DMA,
        ],
        compiler_params=pltpu.CompilerParams(use_tc_tiling_on_sc=False),
    )
    def k(ids_hbm, seg_hbm, pos_hbm, tok_tab, comb_tab, out_hbm,
          idx_tok, seg_v, pos_v, cidx, rows_a, rows_b, sem_a, sem_b):
        wid = lax.axis_index("s") * nc + lax.axis_index("c")
        base = wid * per_w

        def body(g, carry):
            off = base + g * chunk
            pltpu.sync_copy(ids_hbm.at[pl.ds(off, chunk)], idx_tok)
            pltpu.sync_copy(seg_hbm.at[pl.ds(off, chunk)], seg_v)
            pltpu.sync_copy(pos_hbm.at[pl.ds(off, chunk)], pos_v)
            for i in range(chunk // _LANES):
                sl = pl.ds(i * _LANES, _LANES)
                cidx[sl] = seg_v[sl] * seq_vocab + pos_v[sl]
            ca = pltpu.async_copy(tok_tab.at[idx_tok], rows_a, sem_a)
            ca.wait()
            cb = pltpu.async_copy(comb_tab.at[cidx], rows_a, sem_b, add=True)
            cb.wait()
            pltpu.sync_copy(rows_a, out_hbm.at[pl.ds(off, chunk)])
            return carry

        lax.fori_loop(0, n_chunks, body, 0)

    return k


def kernel(input_ids, segment_ids, position_ids, token_table,
           segment_table, position_table):
    b, s = input_ids.shape
    d = token_table.shape[1]
    seq_vocab = position_table.shape[0]
    n = b * s

    if position_ids is None:
        position_ids = jnp.broadcast_to(
            jnp.arange(s, dtype=input_ids.dtype)[None, :], (b, s))

    ids = input_ids.reshape(n).astype(jnp.int32)
    seg = segment_ids.reshape(n).astype(jnp.int32)
    pos = position_ids.reshape(n).astype(jnp.int32)
    comb = (segment_table[:, None, :] + position_table[None, :, :]).reshape(
        segment_table.shape[0] * seq_vocab, d)

    out = _embed_kernel(n, d, seq_vocab)(ids, seg, pos, token_table, comb)
    return out.reshape(b, s, d)


# speedup vs baseline: 6.5729x; 1.2222x over previous
import functools

import jax
import jax.numpy as jnp
from jax import lax
from jax.experimental import pallas as pl
from jax.experimental.pallas import tpu as pltpu
from jax.experimental.pallas import tpu_sc as plsc

_LANES = 16
_NSLOT = 4


def _embed_kernel(n_tokens, d, seq_vocab):
    info = plsc.get_sparse_core_info()
    nc, ns = info.num_cores, info.num_subcores
    nw = nc * ns
    chunk = 80
    per_w = n_tokens // nw
    n_chunks = per_w // chunk
    assert per_w % chunk == 0 and n_tokens % nw == 0
    assert n_chunks % _NSLOT == 0 and n_chunks >= 2 * _NSLOT

    mesh = plsc.VectorSubcoreMesh(core_axis_name="c", subcore_axis_name="s")

    @functools.partial(
        pl.kernel,
        mesh=mesh,
        out_type=jax.ShapeDtypeStruct((n_tokens, d), jnp.float32),
        scratch_types=[
            pltpu.VMEM((_NSLOT, chunk), jnp.int32),
            pltpu.VMEM((_NSLOT, chunk), jnp.int32),
            pltpu.VMEM((_NSLOT, chunk), jnp.int32),
            pltpu.VMEM((_NSLOT, chunk), jnp.int32),
            pltpu.VMEM((_NSLOT, chunk, d), jnp.float32),
            pltpu.SemaphoreType.DMA((_NSLOT,)),
            pltpu.SemaphoreType.DMA((_NSLOT,)),
            pltpu.SemaphoreType.DMA((_NSLOT,)),
            pltpu.SemaphoreType.DMA((_NSLOT,)),
        ],
        compiler_params=pltpu.CompilerParams(use_tc_tiling_on_sc=False),
    )
    def k(ids_hbm, seg_hbm, pos_hbm, tok_tab, comb_tab, out_hbm,
          idxs, segs, poss, cidxs, rows, sem_idx, sem_tok, sem_comb, sem_out):
        wid = lax.axis_index("s") * nc + lax.axis_index("c")
        base = wid * per_w

        def load_idx(g, b):
            off = base + g * chunk
            pltpu.async_copy(ids_hbm.at[pl.ds(off, chunk)], idxs.at[b],
                             sem_idx.at[b])
            pltpu.async_copy(seg_hbm.at[pl.ds(off, chunk)], segs.at[b],
                             sem_idx.at[b])
            pltpu.async_copy(pos_hbm.at[pl.ds(off, chunk)], poss.at[b],
                             sem_idx.at[b])

        def wait_idx(b):
            pltpu.make_async_copy(ids_hbm.at[pl.ds(base, chunk)], idxs.at[b],
                                  sem_idx.at[b]).wait()
            pltpu.make_async_copy(seg_hbm.at[pl.ds(base, chunk)], segs.at[b],
                                  sem_idx.at[b]).wait()
            pltpu.make_async_copy(pos_hbm.at[pl.ds(base, chunk)], poss.at[b],
                                  sem_idx.at[b]).wait()

        def compute_cidx(b):
            for i in range(chunk // _LANES):
                sl = pl.ds(i * _LANES, _LANES)
                cidxs[b, sl] = segs[b, sl] * seq_vocab + poss[b, sl]

        def fire_tok(b):
            pltpu.async_copy(tok_tab.at[idxs.at[b]], rows.at[b],
                             sem_tok.at[b])

        def wait_tok(b):
            pltpu.make_async_copy(tok_tab.at[idxs.at[b]], rows.at[b],
                                  sem_tok.at[b]).wait()

        def fire_comb(b):
            pltpu.async_copy(comb_tab.at[cidxs.at[b]], rows.at[b],
                             sem_comb.at[b], add=True)

        def wait_comb(b):
            pltpu.make_async_copy(comb_tab.at[cidxs.at[b]], rows.at[b],
                                  sem_comb.at[b]).wait()

        def fire_out(g, b):
            off = base + g * chunk
            pltpu.async_copy(rows.at[b], out_hbm.at[pl.ds(off, chunk)],
                             sem_out.at[b])

        def wait_out(b):
            pltpu.make_async_copy(rows.at[b], out_hbm.at[pl.ds(base, chunk)],
                                  sem_out.at[b]).wait()

        load_idx(0, 0)
        load_idx(1, 1)
        load_idx(2, 2)
        wait_idx(0)
        compute_cidx(0)
        fire_tok(0)
        wait_idx(1)
        compute_cidx(1)
        fire_tok(1)
        wait_tok(0)
        fire_comb(0)

        def quad(g4, carry):
            for b in range(_NSLOT):
                g = g4 * _NSLOT + b
                c1 = g + 2

                @pl.when(c1 < n_chunks)
                def _():
                    b1 = (b + 2) % _NSLOT
                    wait_idx(b1)
                    compute_cidx(b1)

                    @pl.when(c1 >= _NSLOT)
                    def _():
                        wait_out(b1)

                    fire_tok(b1)

                @pl.when(g + 1 < n_chunks)
                def _():
                    b2 = (b + 1) % _NSLOT
                    wait_tok(b2)
                    fire_comb(b2)

                wait_comb(b)
                fire_out(g, b)

                @pl.when(g + 3 < n_chunks)
                def _():
                    load_idx(g + 3, (b + 3) % _NSLOT)
            return carry

        lax.fori_loop(0, n_chunks // _NSLOT, quad, 0)
        for b in range(_NSLOT):
            wait_out(b)

    return k


def kernel(input_ids, segment_ids, position_ids, token_table,
           segment_table, position_table):
    b, s = input_ids.shape
    d = token_table.shape[1]
    seq_vocab = position_table.shape[0]
    n = b * s

    if position_ids is None:
        position_ids = jnp.broadcast_to(
            jnp.arange(s, dtype=input_ids.dtype)[None, :], (b, s))

    ids = input_ids.reshape(n).astype(jnp.int32)
    seg = segment_ids.reshape(n).astype(jnp.int32)
    pos = position_ids.reshape(n).astype(jnp.int32)
    comb = (segment_table[:, None, :] + position_table[None, :, :]).reshape(
        segment_table.shape[0] * seq_vocab, d)

    out = _embed_kernel(n, d, seq_vocab)(ids, seg, pos, token_table, comb)
    return out.reshape(b, s, d)


# speedup vs baseline: 6.5940x; 1.0032x over previous
import functools

import jax
import jax.numpy as jnp
from jax import lax
from jax.experimental import pallas as pl
from jax.experimental.pallas import tpu as pltpu
from jax.experimental.pallas import tpu_sc as plsc

_LANES = 16
_NSLOT = 5


def _embed_kernel(n_tokens, d, seq_vocab):
    info = plsc.get_sparse_core_info()
    nc, ns = info.num_cores, info.num_subcores
    nw = nc * ns
    chunk = 128
    per_w = n_tokens // nw
    n_chunks = per_w // chunk
    assert per_w % chunk == 0 and n_tokens % nw == 0
    assert n_chunks % _NSLOT == 0 and n_chunks >= 2 * _NSLOT

    mesh = plsc.VectorSubcoreMesh(core_axis_name="c", subcore_axis_name="s")

    @functools.partial(
        pl.kernel,
        mesh=mesh,
        out_type=jax.ShapeDtypeStruct((n_tokens, d), jnp.float32),
        scratch_types=[
            pltpu.VMEM((_NSLOT, chunk), jnp.int32),
            pltpu.VMEM((_NSLOT, chunk), jnp.int32),
            pltpu.VMEM((_NSLOT, chunk), jnp.int32),
            pltpu.VMEM((_NSLOT, chunk), jnp.int32),
            pltpu.VMEM((_NSLOT, chunk, d), jnp.float32),
            pltpu.SemaphoreType.DMA((_NSLOT,)),
            pltpu.SemaphoreType.DMA((_NSLOT,)),
            pltpu.SemaphoreType.DMA((_NSLOT,)),
            pltpu.SemaphoreType.DMA((_NSLOT,)),
        ],
        compiler_params=pltpu.CompilerParams(use_tc_tiling_on_sc=False),
    )
    def k(ids_hbm, seg_hbm, pos_hbm, tok_tab, comb_tab, out_hbm,
          idxs, segs, poss, cidxs, rows, sem_idx, sem_tok, sem_comb, sem_out):
        wid = lax.axis_index("s") * nc + lax.axis_index("c")
        base = wid * per_w

        def load_idx(g, b):
            off = base + g * chunk
            pltpu.async_copy(ids_hbm.at[pl.ds(off, chunk)], idxs.at[b],
                             sem_idx.at[b])
            pltpu.async_copy(seg_hbm.at[pl.ds(off, chunk)], segs.at[b],
                             sem_idx.at[b])
            pltpu.async_copy(pos_hbm.at[pl.ds(off, chunk)], poss.at[b],
                             sem_idx.at[b])

        def wait_idx(b):
            pltpu.make_async_copy(ids_hbm.at[pl.ds(base, chunk)], idxs.at[b],
                                  sem_idx.at[b]).wait()
            pltpu.make_async_copy(seg_hbm.at[pl.ds(base, chunk)], segs.at[b],
                                  sem_idx.at[b]).wait()
            pltpu.make_async_copy(pos_hbm.at[pl.ds(base, chunk)], poss.at[b],
                                  sem_idx.at[b]).wait()

        def compute_cidx(b):
            for i in range(chunk // _LANES):
                sl = pl.ds(i * _LANES, _LANES)
                cidxs[b, sl] = segs[b, sl] * seq_vocab + poss[b, sl]

        def fire_tok(b):
            pltpu.async_copy(tok_tab.at[idxs.at[b]], rows.at[b],
                             sem_tok.at[b])

        def wait_tok(b):
            pltpu.make_async_copy(tok_tab.at[idxs.at[b]], rows.at[b],
                                  sem_tok.at[b]).wait()

        def fire_comb(b):
            pltpu.async_copy(comb_tab.at[cidxs.at[b]], rows.at[b],
                             sem_comb.at[b], add=True)

        def wait_comb(b):
            pltpu.make_async_copy(comb_tab.at[cidxs.at[b]], rows.at[b],
                                  sem_comb.at[b]).wait()

        def fire_out(g, b):
            off = base + g * chunk
            pltpu.async_copy(rows.at[b], out_hbm.at[pl.ds(off, chunk)],
                             sem_out.at[b])

        def wait_out(b):
            pltpu.make_async_copy(rows.at[b], out_hbm.at[pl.ds(base, chunk)],
                                  sem_out.at[b]).wait()

        load_idx(0, 0)
        load_idx(1, 1)
        load_idx(2, 2)
        wait_idx(0)
        compute_cidx(0)
        fire_tok(0)
        wait_idx(1)
        compute_cidx(1)
        fire_tok(1)
        wait_tok(0)
        fire_comb(0)

        def quad(g4, carry):
            for b in range(_NSLOT):
                g = g4 * _NSLOT + b
                c1 = g + 2

                @pl.when(c1 < n_chunks)
                def _():
                    b1 = (b + 2) % _NSLOT
                    wait_idx(b1)
                    compute_cidx(b1)

                    @pl.when(c1 >= _NSLOT)
                    def _():
                        wait_out(b1)

                    fire_tok(b1)

                @pl.when(g + 1 < n_chunks)
                def _():
                    b2 = (b + 1) % _NSLOT
                    wait_tok(b2)
                    fire_comb(b2)

                wait_comb(b)
                fire_out(g, b)

                @pl.when(g + 3 < n_chunks)
                def _():
                    load_idx(g + 3, (b + 3) % _NSLOT)
            return carry

        lax.fori_loop(0, n_chunks // _NSLOT, quad, 0)
        for b in range(_NSLOT):
            wait_out(b)

    return k


def kernel(input_ids, segment_ids, position_ids, token_table,
           segment_table, position_table):
    b, s = input_ids.shape
    d = token_table.shape[1]
    seq_vocab = position_table.shape[0]
    n = b * s

    if position_ids is None:
        position_ids = jnp.broadcast_to(
            jnp.arange(s, dtype=input_ids.dtype)[None, :], (b, s))

    ids = input_ids.reshape(n).astype(jnp.int32)
    seg = segment_ids.reshape(n).astype(jnp.int32)
    pos = position_ids.reshape(n).astype(jnp.int32)
    comb = (segment_table[:, None, :] + position_table[None, :, :]).reshape(
        segment_table.shape[0] * seq_vocab, d)

    out = _embed_kernel(n, d, seq_vocab)(ids, seg, pos, token_table, comb)
    return out.reshape(b, s, d)
